# Initial kernel scaffold; baseline (speedup 1.0000x reference)
import functools

import jax
import jax.numpy as jnp
from jax.experimental import pallas as pl

_BS = 7
_PAD = _BS - 1


def _dilate1d(v, axis):
    zshape = list(v.shape)
    zshape[axis] = _PAD
    z = jnp.zeros(zshape, v.dtype)
    p = jnp.concatenate([z, v, z], axis=axis)

    def sl(a, start, length):
        return jax.lax.slice_in_dim(a, start, start + length, axis=axis)

    n = p.shape[axis]
    a = jnp.maximum(sl(p, 0, n - 1), sl(p, 1, n - 1))
    b = jnp.maximum(sl(a, 0, n - 3), sl(a, 2, n - 3))
    c = jnp.maximum(sl(b, 0, n - 6), sl(b, 3, n - 6))
    return c


def _mask_kernel(mask_ref, cnt_ref, dil_ref):
    m = mask_ref[...]
    t = _dilate1d(m, axis=2)
    d = _dilate1d(t, axis=1)
    cnt_ref[0, 0, :] = jnp.sum(d, axis=(1, 2))
    dil_ref[...] = d


def _apply_kernel(total, cnt_ref, x_ref, dil_ref, out_ref):
    dropped = jnp.sum(cnt_ref[...])
    scale = total / (total - dropped)
    out_ref[...] = x_ref[...] * ((1.0 - dil_ref[...]) * scale)


@jax.jit
def kernel(x, mask):
    B, C, H, W = x.shape
    Hm, Wm = mask.shape[2], mask.shape[3]
    N = B * C
    CB = 8
    G = N // CB
    x3 = x.reshape(N, H, W)
    m3 = mask.reshape(N, Hm, Wm)

    cnt, dil = pl.pallas_call(
        _mask_kernel,
        grid=(G,),
        in_specs=[pl.BlockSpec((CB, Hm, Wm), lambda i: (i, 0, 0))],
        out_specs=[
            pl.BlockSpec((1, 1, CB), lambda i: (i, 0, 0)),
            pl.BlockSpec((CB, H, W), lambda i: (i, 0, 0)),
        ],
        out_shape=[
            jax.ShapeDtypeStruct((G, 1, CB), jnp.float32),
            jax.ShapeDtypeStruct((N, H, W), jnp.float32),
        ],
    )(m3)

    out = pl.pallas_call(
        functools.partial(_apply_kernel, jnp.float32(x.size)),
        grid=(G,),
        in_specs=[
            pl.BlockSpec((G, 1, CB), lambda i: (0, 0, 0)),
            pl.BlockSpec((CB, H, W), lambda i: (i, 0, 0)),
            pl.BlockSpec((CB, H, W), lambda i: (i, 0, 0)),
        ],
        out_specs=pl.BlockSpec((CB, H, W), lambda i: (i, 0, 0)),
        out_shape=jax.ShapeDtypeStruct((N, H, W), jnp.float32),
    )(cnt, x3, dil)

    return out.reshape(B, C, H, W)


# speedup vs baseline: 2.0586x; 2.0586x over previous
import functools

import jax
import jax.numpy as jnp
from jax.experimental import pallas as pl

_BS = 7
_PAD = _BS - 1


def _dilate1d(v, axis):
    zshape = list(v.shape)
    zshape[axis] = _PAD
    z = jnp.zeros(zshape, v.dtype)
    p = jnp.concatenate([z, v, z], axis=axis)

    def sl(a, start, length):
        return jax.lax.slice_in_dim(a, start, start + length, axis=axis)

    n = p.shape[axis]
    a = jnp.maximum(sl(p, 0, n - 1), sl(p, 1, n - 1))
    b = jnp.maximum(sl(a, 0, n - 3), sl(a, 2, n - 3))
    c = jnp.maximum(sl(b, 0, n - 6), sl(b, 3, n - 6))
    return c


def _mask_kernel(mask_ref, cnt_ref, dil_ref):
    m = mask_ref[...]
    t = _dilate1d(m, axis=2)
    d = _dilate1d(t, axis=1)
    cnt_ref[0, 0, :] = jnp.sum(d, axis=(1, 2))
    dil_ref[...] = d


def _apply_kernel(total, cnt_ref, x_ref, dil_ref, out_ref):
    dropped = jnp.sum(cnt_ref[...])
    scale = jnp.float32(total) / (jnp.float32(total) - dropped)
    out_ref[...] = x_ref[...] * ((1.0 - dil_ref[...]) * scale)


@jax.jit
def kernel(x, mask):
    B, C, H, W = x.shape
    Hm, Wm = mask.shape[2], mask.shape[3]
    N = B * C
    CB = 8
    G = N // CB
    x3 = x.reshape(N, H, W)
    m3 = mask.reshape(N, Hm, Wm)

    cnt, dil = pl.pallas_call(
        _mask_kernel,
        grid=(G,),
        in_specs=[pl.BlockSpec((CB, Hm, Wm), lambda i: (i, 0, 0))],
        out_specs=[
            pl.BlockSpec((1, 1, CB), lambda i: (i, 0, 0)),
            pl.BlockSpec((CB, H, W), lambda i: (i, 0, 0)),
        ],
        out_shape=[
            jax.ShapeDtypeStruct((G, 1, CB), jnp.float32),
            jax.ShapeDtypeStruct((N, H, W), jnp.float32),
        ],
    )(m3)

    out = pl.pallas_call(
        functools.partial(_apply_kernel, float(x.size)),
        grid=(G,),
        in_specs=[
            pl.BlockSpec((G, 1, CB), lambda i: (0, 0, 0)),
            pl.BlockSpec((CB, H, W), lambda i: (i, 0, 0)),
            pl.BlockSpec((CB, H, W), lambda i: (i, 0, 0)),
        ],
        out_specs=pl.BlockSpec((CB, H, W), lambda i: (i, 0, 0)),
        out_shape=jax.ShapeDtypeStruct((N, H, W), jnp.float32),
    )(cnt, x3, dil)

    return out.reshape(B, C, H, W)


# speedup vs baseline: 2.0799x; 1.0103x over previous
import functools

import jax
import jax.numpy as jnp
from jax.experimental import pallas as pl

_BS = 7
_PAD = _BS - 1


def _dilate1d(v, axis):
    zshape = list(v.shape)
    zshape[axis] = _PAD
    z = jnp.zeros(zshape, v.dtype)
    p = jnp.concatenate([z, v, z], axis=axis)

    def sl(a, start, length):
        return jax.lax.slice_in_dim(a, start, start + length, axis=axis)

    n = p.shape[axis]
    a = jnp.maximum(sl(p, 0, n - 1), sl(p, 1, n - 1))
    b = jnp.maximum(sl(a, 0, n - 3), sl(a, 2, n - 3))
    c = jnp.maximum(sl(b, 0, n - 6), sl(b, 3, n - 6))
    return c


def _mask_kernel(mask_ref, cnt_ref, dil_ref):
    m = mask_ref[...]
    t = _dilate1d(m, axis=2)
    d = _dilate1d(t, axis=1)
    cnt_ref[0, 0, :] = jnp.sum(d, axis=(1, 2))
    dil_ref[...] = d.astype(jnp.int8)


def _apply_kernel(total, cnt_ref, x_ref, dil_ref, out_ref):
    dropped = jnp.sum(cnt_ref[...])
    scale = jnp.float32(total) / (jnp.float32(total) - dropped)
    out_ref[...] = x_ref[...] * (
        (1.0 - dil_ref[...].astype(jnp.float32)) * scale)


@jax.jit
def kernel(x, mask):
    B, C, H, W = x.shape
    Hm, Wm = mask.shape[2], mask.shape[3]
    N = B * C
    CB = 8
    G = N // CB
    x3 = x.reshape(N, H, W)
    m3 = mask.reshape(N, Hm, Wm)

    cnt, dil = pl.pallas_call(
        _mask_kernel,
        grid=(G,),
        in_specs=[pl.BlockSpec((CB, Hm, Wm), lambda i: (i, 0, 0))],
        out_specs=[
            pl.BlockSpec((1, 1, CB), lambda i: (i, 0, 0)),
            pl.BlockSpec((CB, H, W), lambda i: (i, 0, 0)),
        ],
        out_shape=[
            jax.ShapeDtypeStruct((G, 1, CB), jnp.float32),
            jax.ShapeDtypeStruct((N, H, W), jnp.int8),
        ],
    )(m3)

    out = pl.pallas_call(
        functools.partial(_apply_kernel, float(x.size)),
        grid=(G,),
        in_specs=[
            pl.BlockSpec((G, 1, CB), lambda i: (0, 0, 0)),
            pl.BlockSpec((CB, H, W), lambda i: (i, 0, 0)),
            pl.BlockSpec((CB, H, W), lambda i: (i, 0, 0)),
        ],
        out_specs=pl.BlockSpec((CB, H, W), lambda i: (i, 0, 0)),
        out_shape=jax.ShapeDtypeStruct((N, H, W), jnp.float32),
    )(cnt, x3, dil)

    return out.reshape(B, C, H, W)
